# baseline (speedup 1.0000x reference)
import functools

import jax
import jax.numpy as jnp
from jax import lax
from jax.experimental import pallas as pl
from jax.experimental.pallas import tpu as pltpu
from jax.experimental.pallas import tpu_sc as plsc

BS, SEQ1, D = 4, 8193, 768
TOT = BS * D
NT = D // 128
_INFO = plsc.get_sparse_core_info()
NC, NS, L = _INFO.num_cores, _INFO.num_subcores, _INFO.num_lanes
NW = NC * NS
PER_W = TOT // NW
JW = D // NW

_mesh = plsc.VectorSubcoreMesh(core_axis_name="c", subcore_axis_name="s")


@functools.partial(
    pl.kernel,
    mesh=_mesh,
    out_type=jax.ShapeDtypeStruct((TOT,), jnp.float32),
    scratch_types=[
        pltpu.VMEM((JW,), jnp.int32),
        pltpu.VMEM((JW,), jnp.int32),
        pltpu.VMEM((JW, BS, 256), jnp.float32),
        pltpu.VMEM((PER_W,), jnp.float32),
        pltpu.SemaphoreType.DMA,
    ],
    compiler_params=pltpu.CompilerParams(
        use_tc_tiling_on_sc=True, needs_layout_passes=False),
)
def _sc_gather(tok_hbm, idx_hbm, out_hbm, idx_v, row_v, slab_v, val_v, sem):
    wid = lax.axis_index("s") * NC + lax.axis_index("c")
    jbase = wid * JW
    ab = pl.multiple_of(jnp.minimum((jbase >> 7) << 7, D - 256), 128)
    pltpu.sync_copy(idx_hbm.at[pl.ds(jbase, JW)], idx_v)
    lane = lax.iota(jnp.int32, L)
    for st in (0, JW - L):
        row_v[pl.ds(st, L)] = idx_v[pl.ds(st, L)] + 1
    pltpu.async_copy(tok_hbm.at[row_v, :, pl.ds(ab, 256)], slab_v, sem).wait()
    for i in range(PER_W // L):
        e = i * L + lane
        b_v = e // JW
        k_v = e - b_v * JW
        val_v[pl.ds(i * L, L)] = plsc.load_gather(
            slab_v, [k_v, b_v, jbase + k_v - ab])
    for b in range(BS):
        pltpu.sync_copy(val_v.at[pl.ds(b * JW, JW)],
                        out_hbm.at[pl.ds(b * D + jbase, JW)])


def _tc_q_body(cls_ref, win_ref, q_ref):
    q_ref[...] = lax.dot_general(
        cls_ref[...], win_ref[...], (((1,), (1,)), ((), ())),
        precision=lax.Precision.HIGHEST, preferred_element_type=jnp.float32)


_tc_q = pl.pallas_call(
    _tc_q_body,
    grid=(NT,),
    out_shape=jax.ShapeDtypeStruct((BS, D), jnp.float32),
    in_specs=[
        pl.BlockSpec((BS, D), lambda i: (0, 0)),
        pl.BlockSpec((128, D), lambda i: (i, 0)),
    ],
    out_specs=pl.BlockSpec((BS, 128), lambda i: (0, i)),
)


def _tc_out_body(emb_ref, q_ref, wout_ref, out_ref, comb_ref):
    i = pl.program_id(0)

    @pl.when(i == 0)
    def _():
        emb = jnp.concatenate(
            [emb_ref[pl.ds(b * D, D)].reshape(1, D) for b in range(BS)],
            axis=0)
        comb_ref[...] = jnp.concatenate([emb, q_ref[...]], axis=1)

    out_ref[...] = jnp.tanh(lax.dot_general(
        comb_ref[...], wout_ref[...], (((1,), (1,)), ((), ())),
        precision=lax.Precision.HIGHEST, preferred_element_type=jnp.float32))


_tc_out = pl.pallas_call(
    _tc_out_body,
    grid=(NT,),
    out_shape=jax.ShapeDtypeStruct((BS, D), jnp.float32),
    in_specs=[
        pl.BlockSpec((TOT,), lambda i: (0,)),
        pl.BlockSpec((BS, D), lambda i: (0, 0)),
        pl.BlockSpec((128, 2 * D), lambda i: (i, 0)),
    ],
    out_specs=pl.BlockSpec((BS, 128), lambda i: (0, i)),
    scratch_shapes=[pltpu.VMEM((BS, 2 * D), jnp.float32)],
)


def kernel(output_tokens_from_bert, cls_output, embrace_idx, W_in, W_out):
    tok_t = jnp.swapaxes(output_tokens_from_bert, 0, 1)
    emb_flat = _sc_gather(tok_t, embrace_idx)
    q = _tc_q(cls_output, W_in)
    return _tc_out(emb_flat, q, W_out)


# speedup vs baseline: 1.0737x; 1.0737x over previous
import functools

import jax
import jax.numpy as jnp
from jax import lax
from jax.experimental import pallas as pl
from jax.experimental.pallas import tpu as pltpu
from jax.experimental.pallas import tpu_sc as plsc

BS, SEQ1, D = 4, 8193, 768
TOT = BS * D
NT = D // 128
_INFO = plsc.get_sparse_core_info()
NC, NS, L = _INFO.num_cores, _INFO.num_subcores, _INFO.num_lanes
NW = NC * NS
PER_W = TOT // NW
JW = D // NW

_mesh = plsc.VectorSubcoreMesh(core_axis_name="c", subcore_axis_name="s")


@functools.partial(
    pl.kernel,
    mesh=_mesh,
    out_type=jax.ShapeDtypeStruct((TOT,), jnp.float32),
    scratch_types=[
        pltpu.VMEM((JW,), jnp.int32),
        pltpu.VMEM((JW,), jnp.int32),
        pltpu.VMEM((JW, BS, 256), jnp.float32),
        pltpu.VMEM((PER_W,), jnp.float32),
        pltpu.SemaphoreType.DMA,
    ],
    compiler_params=pltpu.CompilerParams(
        use_tc_tiling_on_sc=True, needs_layout_passes=False),
)
def _sc_gather(tok_hbm, idx_hbm, out_hbm, idx_v, row_v, slab_v, val_v, sem):
    wid = lax.axis_index("s") * NC + lax.axis_index("c")
    jbase = wid * JW
    ab = pl.multiple_of(jnp.minimum((jbase >> 7) << 7, D - 256), 128)
    pltpu.sync_copy(idx_hbm.at[pl.ds(jbase, JW)], idx_v)
    lane = lax.iota(jnp.int32, L)
    for st in (0, JW - L):
        row_v[pl.ds(st, L)] = idx_v[pl.ds(st, L)] + 1
    pltpu.async_copy(tok_hbm.at[row_v, :, pl.ds(ab, 256)], slab_v, sem).wait()
    for i in range(PER_W // L):
        e = i * L + lane
        b_v = e // JW
        k_v = e - b_v * JW
        val_v[pl.ds(i * L, L)] = plsc.load_gather(
            slab_v, [k_v, b_v, jbase + k_v - ab])
    for b in range(BS):
        pltpu.sync_copy(val_v.at[pl.ds(b * JW, JW)],
                        out_hbm.at[pl.ds(b * D + jbase, JW)])


def _tc_q_body(cls_ref, win_ref, woutb_ref, h2_ref):
    q = lax.dot_general(
        cls_ref[...], win_ref[...], (((1,), (1,)), ((), ())),
        precision=lax.Precision.HIGHEST, preferred_element_type=jnp.float32)
    h2_ref[...] = lax.dot_general(
        q, woutb_ref[...], (((1,), (1,)), ((), ())),
        precision=lax.Precision.HIGHEST, preferred_element_type=jnp.float32)


_tc_q = pl.pallas_call(
    _tc_q_body,
    grid=(1,),
    out_shape=jax.ShapeDtypeStruct((BS, D), jnp.float32),
    in_specs=[
        pl.BlockSpec((BS, D), lambda i: (0, 0)),
        pl.BlockSpec((D, D), lambda i: (0, 0)),
        pl.BlockSpec((D, D), lambda i: (0, 1)),
    ],
    out_specs=pl.BlockSpec((BS, D), lambda i: (0, 0)),
)


def _tc_out_body(emb_ref, h2_ref, wouta_ref, out_ref):
    emb = jnp.concatenate(
        [emb_ref[pl.ds(b * D, D)].reshape(1, D) for b in range(BS)], axis=0)
    out_ref[...] = jnp.tanh(h2_ref[...] + lax.dot_general(
        emb, wouta_ref[...], (((1,), (1,)), ((), ())),
        precision=lax.Precision.HIGHEST, preferred_element_type=jnp.float32))


_tc_out = pl.pallas_call(
    _tc_out_body,
    grid=(1,),
    out_shape=jax.ShapeDtypeStruct((BS, D), jnp.float32),
    in_specs=[
        pl.BlockSpec((TOT,), lambda i: (0,)),
        pl.BlockSpec((BS, D), lambda i: (0, 0)),
        pl.BlockSpec((D, D), lambda i: (0, 0)),
    ],
    out_specs=pl.BlockSpec((BS, D), lambda i: (0, 0)),
)


def kernel(output_tokens_from_bert, cls_output, embrace_idx, W_in, W_out):
    tok_t = jnp.swapaxes(output_tokens_from_bert, 0, 1)
    emb_flat = _sc_gather(tok_t, embrace_idx)
    h2 = _tc_q(cls_output, W_in, W_out)
    return _tc_out(emb_flat, h2, W_out)


# speedup vs baseline: 1.1716x; 1.0911x over previous
import functools

import jax
import jax.numpy as jnp
from jax import lax
from jax.experimental import pallas as pl
from jax.experimental.pallas import tpu as pltpu
from jax.experimental.pallas import tpu_sc as plsc

BS, SEQ1, D = 4, 8193, 768
TOT = BS * D
NT = D // 128
_INFO = plsc.get_sparse_core_info()
NC, NS, L = _INFO.num_cores, _INFO.num_subcores, _INFO.num_lanes
NW = NC * NS
PER_W = TOT // NW
JW = D // NW

_mesh = plsc.VectorSubcoreMesh(core_axis_name="c", subcore_axis_name="s")


@functools.partial(
    pl.kernel,
    mesh=_mesh,
    out_type=jax.ShapeDtypeStruct((TOT,), jnp.float32),
    scratch_types=[
        pltpu.VMEM((JW,), jnp.int32),
        pltpu.VMEM((JW,), jnp.int32),
        pltpu.VMEM((JW, BS, 256), jnp.float32),
        pltpu.VMEM((PER_W,), jnp.float32),
        pltpu.SemaphoreType.DMA,
    ],
    compiler_params=pltpu.CompilerParams(
        use_tc_tiling_on_sc=True, needs_layout_passes=False),
)
def _sc_gather(tok_hbm, idx_hbm, out_hbm, idx_v, row_v, slab_v, val_v, sem):
    wid = lax.axis_index("s") * NC + lax.axis_index("c")
    jbase = wid * JW
    ab = pl.multiple_of(jnp.minimum((jbase >> 7) << 7, D - 256), 128)
    pltpu.sync_copy(idx_hbm.at[pl.ds(jbase, JW)], idx_v)
    lane = lax.iota(jnp.int32, L)
    for st in (0, JW - L):
        row_v[pl.ds(st, L)] = idx_v[pl.ds(st, L)] + 1
    pltpu.async_copy(tok_hbm.at[row_v, :, pl.ds(ab, 256)], slab_v, sem).wait()
    for i in range(PER_W // L):
        e = i * L + lane
        b_v = e // JW
        k_v = e - b_v * JW
        val_v[pl.ds(i * L, L)] = plsc.load_gather(
            slab_v, [k_v, b_v, jbase + k_v - ab])
    for b in range(BS):
        pltpu.sync_copy(val_v.at[pl.ds(b * JW, JW)],
                        out_hbm.at[pl.ds(b * D + jbase, JW)])


def _tc_q_body(cls_ref, win_ref, woutb_ref, h2_ref):
    q = lax.dot_general(
        cls_ref[...], win_ref[...], (((1,), (1,)), ((), ())),
        preferred_element_type=jnp.float32)
    h2_ref[...] = lax.dot_general(
        q, woutb_ref[...], (((1,), (1,)), ((), ())),
        preferred_element_type=jnp.float32)


_tc_q = pl.pallas_call(
    _tc_q_body,
    grid=(1,),
    out_shape=jax.ShapeDtypeStruct((BS, D), jnp.float32),
    in_specs=[
        pl.BlockSpec((BS, D), lambda i: (0, 0)),
        pl.BlockSpec((D, D), lambda i: (0, 0)),
        pl.BlockSpec((D, D), lambda i: (0, 1)),
    ],
    out_specs=pl.BlockSpec((BS, D), lambda i: (0, 0)),
)


def _tc_out_body(emb_ref, h2_ref, wouta_ref, out_ref):
    emb = jnp.concatenate(
        [emb_ref[pl.ds(b * D, D)].reshape(1, D) for b in range(BS)], axis=0)
    out_ref[...] = jnp.tanh(h2_ref[...] + lax.dot_general(
        emb, wouta_ref[...], (((1,), (1,)), ((), ())),
        preferred_element_type=jnp.float32))


_tc_out = pl.pallas_call(
    _tc_out_body,
    grid=(1,),
    out_shape=jax.ShapeDtypeStruct((BS, D), jnp.float32),
    in_specs=[
        pl.BlockSpec((TOT,), lambda i: (0,)),
        pl.BlockSpec((BS, D), lambda i: (0, 0)),
        pl.BlockSpec((D, D), lambda i: (0, 0)),
    ],
    out_specs=pl.BlockSpec((BS, D), lambda i: (0, 0)),
)


def kernel(output_tokens_from_bert, cls_output, embrace_idx, W_in, W_out):
    tok_t = jnp.swapaxes(output_tokens_from_bert, 0, 1)
    emb_flat = _sc_gather(tok_t, embrace_idx)
    h2 = _tc_q(cls_output, W_in, W_out)
    return _tc_out(emb_flat, h2, W_out)


# speedup vs baseline: 1.1774x; 1.0050x over previous
import functools

import jax
import jax.numpy as jnp
from jax import lax
from jax.experimental import pallas as pl
from jax.experimental.pallas import tpu as pltpu
from jax.experimental.pallas import tpu_sc as plsc

BS, SEQ1, D = 4, 8193, 768
TOT = BS * D
NT = D // 128
_INFO = plsc.get_sparse_core_info()
NC, NS, L = _INFO.num_cores, _INFO.num_subcores, _INFO.num_lanes
NW = NC * NS
PER_W = TOT // NW
JW = D // NW

_mesh = plsc.VectorSubcoreMesh(core_axis_name="c", subcore_axis_name="s")


@functools.partial(
    pl.kernel,
    mesh=_mesh,
    out_type=jax.ShapeDtypeStruct((TOT,), jnp.float32),
    scratch_types=[
        pltpu.VMEM((JW,), jnp.int32),
        pltpu.VMEM((JW,), jnp.int32),
        pltpu.VMEM((JW, BS, 256), jnp.float32),
        pltpu.VMEM((PER_W,), jnp.float32),
        pltpu.SemaphoreType.DMA,
    ],
    compiler_params=pltpu.CompilerParams(
        use_tc_tiling_on_sc=True, needs_layout_passes=False),
)
def _sc_gather(tok_hbm, idx_hbm, out_hbm, idx_v, row_v, slab_v, val_v, sem):
    wid = lax.axis_index("s") * NC + lax.axis_index("c")
    jbase = wid * JW
    ab = pl.multiple_of(jnp.minimum((jbase >> 7) << 7, D - 256), 128)
    pltpu.sync_copy(idx_hbm.at[pl.ds(jbase, JW)], idx_v)
    lane = lax.iota(jnp.int32, L)
    for st in (0, JW - L):
        row_v[pl.ds(st, L)] = idx_v[pl.ds(st, L)] + 1
    pltpu.async_copy(tok_hbm.at[row_v, :, pl.ds(ab, 256)], slab_v, sem).wait()
    for i in range(PER_W // L):
        e = i * L + lane
        b_v = e // JW
        k_v = e - b_v * JW
        val_v[pl.ds(i * L, L)] = plsc.load_gather(
            slab_v, [k_v, b_v, jbase + k_v - ab])
    stores = [pltpu.async_copy(val_v.at[pl.ds(b * JW, JW)],
                               out_hbm.at[pl.ds(b * D + jbase, JW)], sem)
              for b in range(BS)]
    for c in stores:
        c.wait()


def _tc_q_body(cls_ref, win_ref, woutb_ref, h2_ref):
    q = lax.dot_general(
        cls_ref[...], win_ref[...], (((1,), (1,)), ((), ())),
        preferred_element_type=jnp.float32)
    h2_ref[...] = lax.dot_general(
        q, woutb_ref[...], (((1,), (1,)), ((), ())),
        preferred_element_type=jnp.float32)


_tc_q = pl.pallas_call(
    _tc_q_body,
    grid=(1,),
    out_shape=jax.ShapeDtypeStruct((BS, D), jnp.float32),
    in_specs=[
        pl.BlockSpec((BS, D), lambda i: (0, 0)),
        pl.BlockSpec((D, D), lambda i: (0, 0)),
        pl.BlockSpec((D, D), lambda i: (0, 1)),
    ],
    out_specs=pl.BlockSpec((BS, D), lambda i: (0, 0)),
)


def _tc_out_body(emb_ref, h2_ref, wouta_ref, out_ref):
    emb = jnp.concatenate(
        [emb_ref[pl.ds(b * D, D)].reshape(1, D) for b in range(BS)], axis=0)
    out_ref[...] = jnp.tanh(h2_ref[...] + lax.dot_general(
        emb, wouta_ref[...], (((1,), (1,)), ((), ())),
        preferred_element_type=jnp.float32))


_tc_out = pl.pallas_call(
    _tc_out_body,
    grid=(1,),
    out_shape=jax.ShapeDtypeStruct((BS, D), jnp.float32),
    in_specs=[
        pl.BlockSpec((TOT,), lambda i: (0,)),
        pl.BlockSpec((BS, D), lambda i: (0, 0)),
        pl.BlockSpec((D, D), lambda i: (0, 0)),
    ],
    out_specs=pl.BlockSpec((BS, D), lambda i: (0, 0)),
)


def kernel(output_tokens_from_bert, cls_output, embrace_idx, W_in, W_out):
    tok_t = jnp.swapaxes(output_tokens_from_bert, 0, 1)
    emb_flat = _sc_gather(tok_t, embrace_idx)
    h2 = _tc_q(cls_output, W_in, W_out)
    return _tc_out(emb_flat, h2, W_out)
